# baseline (speedup 1.0000x reference)
import functools

import jax
import jax.numpy as jnp
from jax import lax
from jax.experimental import pallas as pl
from jax.experimental.pallas import tpu as pltpu
from jax.experimental.pallas import tpu_sc as plsc

_NUM_CORES = 2
_NUM_SUBCORES = 16
_NUM_WORKERS = _NUM_CORES * _NUM_SUBCORES
_LANES = 16
_SLOTS = 64


def _detile_pair(tableT):
    D, V = tableT.shape
    BM = 8192
    grid = (V + BM - 1) // BM

    def body(t_ref, o_ref):
        x = lax.transpose(t_ref[...], (1, 0))
        x3 = x.reshape(BM // 2, 2, D)
        o_ref[:, 0:D] = x3[:, 0, :]
        o_ref[:, D:2 * D] = x3[:, 1, :]

    return pl.pallas_call(
        body,
        grid=(grid,),
        in_specs=[pl.BlockSpec((D, BM), lambda i: (0, i))],
        out_specs=pl.BlockSpec((BM // 2, 2 * D), lambda i: (i, 0)),
        out_shape=jax.ShapeDtypeStruct((V // 2, 2 * D), jnp.float32),
    )(tableT)


def _embed_bag_mean(textp, table2, B, BAG, D):
    assert B % _NUM_WORKERS == 0
    bpw = B // _NUM_WORKERS
    nch = D // _LANES
    nck = _SLOTS // _LANES

    mesh = plsc.VectorSubcoreMesh(core_axis_name="c", subcore_axis_name="s")

    @functools.partial(
        pl.kernel,
        out_type=jax.ShapeDtypeStruct((B, D), jnp.float32),
        mesh=mesh,
        compiler_params=pltpu.CompilerParams(needs_layout_passes=False),
        scratch_types=[
            pltpu.VMEM((bpw, _SLOTS), jnp.int32),
            pltpu.VMEM((bpw * _SLOTS,), jnp.int32),
            pltpu.VMEM((BAG, 2 * D), jnp.float32),
            pltpu.VMEM((BAG, 2 * D), jnp.float32),
            pltpu.VMEM((bpw, D), jnp.float32),
            pltpu.SMEM((bpw,), jnp.int32),
            pltpu.SemaphoreType.DMA,
            pltpu.SemaphoreType.DMA,
        ],
    )
    def k(text_hbm, table_hbm, out_hbm, idx_v, hidx_v, rows0, rows1, out_v,
          ce_s, sem0, sem1):
        wid = lax.axis_index("s") * _NUM_CORES + lax.axis_index("c")
        base = wid * bpw
        pltpu.sync_copy(text_hbm.at[pl.ds(base, bpw)], idx_v)

        scale = jnp.float32(1.0 / BAG)
        lane = lax.iota(jnp.int32, _LANES)
        one = jnp.ones((_LANES,), jnp.int32)

        def part_body(b, carry):
            vs, pars, cumes, nes = [], [], [], []
            ne_tot = jnp.zeros((_LANES,), jnp.int32)
            for q in range(nck):
                v = idx_v[b, pl.ds(q * _LANES, _LANES)]
                par = v & 1
                meven = one - par
                cume = plsc.cumsum(meven)
                ne = plsc.all_reduce_population_count(par == 0)
                vs.append(v)
                pars.append(par)
                cumes.append(cume)
                nes.append(ne)
                ne_tot = ne_tot + ne
            ce_s[b] = lax.reduce_max(ne_tot, axes=(0,))
            oe = jnp.zeros((_LANES,), jnp.int32)
            oo = ne_tot
            for q in range(nck):
                v, par, cume, ne = vs[q], pars[q], cumes[q], nes[q]
                cumo = (lane + 1) - cume
                pos = jnp.where(par == 0, oe + cume - 1, oo + cumo - 1)
                plsc.store_scatter(
                    hidx_v, [pos + b * _SLOTS],
                    lax.shift_right_logical(v, 1),
                )
                oe = oe + ne
                oo = oo + (_LANES - ne)
            return carry

        lax.fori_loop(0, bpw, part_body, 0)

        def accum(b, rows_v):
            ce = ce_s[b]

            def lo_body(j, accs):
                return tuple(
                    accs[c] + rows_v[j, pl.ds(c * _LANES, _LANES)]
                    for c in range(nch)
                )

            def hi_body(j, accs):
                return tuple(
                    accs[c] + rows_v[j, pl.ds(D + c * _LANES, _LANES)]
                    for c in range(nch)
                )

            zeros = tuple(jnp.zeros((_LANES,), jnp.float32)
                          for _ in range(nch))
            accs = lax.fori_loop(0, ce, lo_body, zeros)
            accs = lax.fori_loop(ce, BAG, hi_body, accs)
            for c in range(nch):
                out_v[b, pl.ds(c * _LANES, _LANES)] = accs[c] * scale

        pltpu.async_copy(table_hbm.at[hidx_v.at[pl.ds(0, BAG)]], rows0, sem0)
        pltpu.async_copy(
            table_hbm.at[hidx_v.at[pl.ds(_SLOTS, BAG)]], rows1, sem1)

        def bag_body(i, carry):
            b0 = 2 * i
            pltpu.make_async_copy(
                table_hbm.at[hidx_v.at[pl.ds(b0 * _SLOTS, BAG)]],
                rows0, sem0).wait()
            accum(b0, rows0)

            @pl.when(i < bpw // 2 - 1)
            def _():
                pltpu.async_copy(
                    table_hbm.at[hidx_v.at[pl.ds((b0 + 2) * _SLOTS, BAG)]],
                    rows0, sem0)

            b1 = b0 + 1
            pltpu.make_async_copy(
                table_hbm.at[hidx_v.at[pl.ds(b1 * _SLOTS, BAG)]],
                rows1, sem1).wait()
            accum(b1, rows1)

            @pl.when(i < bpw // 2 - 1)
            def _():
                pltpu.async_copy(
                    table_hbm.at[hidx_v.at[pl.ds((b1 + 2) * _SLOTS, BAG)]],
                    rows1, sem1)

            return carry

        lax.fori_loop(0, bpw // 2, bag_body, 0)
        pltpu.sync_copy(out_v, out_hbm.at[pl.ds(base, bpw)])

    return k(textp, table2)


def _mlp(x, W1, b1, W2, b2):
    B, D = x.shape
    C = W2.shape[0]
    BM = 512
    assert B % BM == 0

    def body(x_ref, w1_ref, b1_ref, w2_ref, b2_ref, o_ref):
        h = lax.dot_general(
            x_ref[...], w1_ref[...], (((1,), (1,)), ((), ())),
            preferred_element_type=jnp.float32,
        )
        h = jnp.maximum(h + b1_ref[...], 0.0)
        o_ref[...] = lax.dot_general(
            h, w2_ref[...], (((1,), (1,)), ((), ())),
            preferred_element_type=jnp.float32,
        ) + b2_ref[...]

    return pl.pallas_call(
        body,
        grid=(B // BM,),
        in_specs=[
            pl.BlockSpec((BM, D), lambda i: (i, 0)),
            pl.BlockSpec((D, D), lambda i: (0, 0)),
            pl.BlockSpec((1, D), lambda i: (0, 0)),
            pl.BlockSpec((C, D), lambda i: (0, 0)),
            pl.BlockSpec((1, C), lambda i: (0, 0)),
        ],
        out_specs=pl.BlockSpec((BM, C), lambda i: (i, 0)),
        out_shape=jax.ShapeDtypeStruct((B, C), jnp.float32),
    )(x, W1, b1.reshape(1, D), W2, b2.reshape(1, C))


def kernel(text, offsets, table, W1, b1, W2, b2):
    del offsets
    B, BAG = text.shape
    D = table.shape[1]
    table2 = _detile_pair(table.T)
    textp = jnp.pad(text, ((0, 0), (0, _SLOTS - BAG)), constant_values=1)
    pooled = _embed_bag_mean(textp, table2, B, BAG, D)
    return _mlp(pooled, W1, b1, W2, b2)


# speedup vs baseline: 1.4222x; 1.4222x over previous
import functools

import jax
import jax.numpy as jnp
from jax import lax
from jax.experimental import pallas as pl
from jax.experimental.pallas import tpu as pltpu
from jax.experimental.pallas import tpu_sc as plsc

_NUM_CORES = 2
_NUM_SUBCORES = 16
_NUM_WORKERS = _NUM_CORES * _NUM_SUBCORES
_LANES = 16


def _detile(tableT):
    D, V = tableT.shape
    BM = 8192
    grid = (V + BM - 1) // BM

    def body(t_ref, o_ref):
        o_ref[:, 0:D] = lax.transpose(t_ref[...], (1, 0))

    return pl.pallas_call(
        body,
        grid=(grid,),
        in_specs=[pl.BlockSpec((D, BM), lambda i: (0, i))],
        out_specs=pl.BlockSpec((BM, 2 * D), lambda i: (i, 0)),
        out_shape=jax.ShapeDtypeStruct((V, 2 * D), jnp.float32),
    )(tableT)


def _embed_bag_mean(text2, table2, B, BAG, D):
    npair = B // 2
    assert npair % _NUM_WORKERS == 0
    ppw = npair // _NUM_WORKERS
    bpw = 2 * ppw
    nch = D // _LANES
    PB = 2 * BAG

    mesh = plsc.VectorSubcoreMesh(core_axis_name="c", subcore_axis_name="s")

    @functools.partial(
        pl.kernel,
        out_type=jax.ShapeDtypeStruct((B, D), jnp.float32),
        mesh=mesh,
        scratch_types=[
            pltpu.VMEM((ppw, PB), jnp.int32),
            pltpu.VMEM((PB, 2 * D), jnp.float32),
            pltpu.VMEM((PB, 2 * D), jnp.float32),
            pltpu.VMEM((bpw, D), jnp.float32),
            pltpu.SemaphoreType.DMA,
            pltpu.SemaphoreType.DMA,
        ],
    )
    def k(text_hbm, table_hbm, out_hbm, idx_v, rows0, rows1, out_v, sem0, sem1):
        wid = lax.axis_index("s") * _NUM_CORES + lax.axis_index("c")
        base = wid * ppw
        pltpu.sync_copy(text_hbm.at[pl.ds(base, ppw)], idx_v)

        scale = jnp.float32(1.0 / BAG)

        def accum(p, rows_v):
            def j_body(j, accs):
                new = []
                for h in range(2):
                    for c in range(nch):
                        new.append(
                            accs[h * nch + c]
                            + rows_v[h * BAG + j, pl.ds(c * _LANES, _LANES)]
                        )
                return tuple(new)

            accs = lax.fori_loop(
                0, BAG, j_body,
                tuple(jnp.zeros((_LANES,), jnp.float32) for _ in range(2 * nch)),
            )
            for h in range(2):
                for c in range(nch):
                    out_v[2 * p + h, pl.ds(c * _LANES, _LANES)] = (
                        accs[h * nch + c] * scale
                    )

        pltpu.async_copy(table_hbm.at[idx_v.at[0]], rows0, sem0)
        pltpu.async_copy(table_hbm.at[idx_v.at[1]], rows1, sem1)

        def pair_body(i, carry):
            p0 = 2 * i
            pltpu.make_async_copy(table_hbm.at[idx_v.at[p0]], rows0, sem0).wait()
            accum(p0, rows0)

            @pl.when(i < ppw // 2 - 1)
            def _():
                pltpu.async_copy(table_hbm.at[idx_v.at[p0 + 2]], rows0, sem0)

            p1 = p0 + 1
            pltpu.make_async_copy(table_hbm.at[idx_v.at[p1]], rows1, sem1).wait()
            accum(p1, rows1)

            @pl.when(i < ppw // 2 - 1)
            def _():
                pltpu.async_copy(table_hbm.at[idx_v.at[p1 + 2]], rows1, sem1)

            return carry

        lax.fori_loop(0, ppw // 2, pair_body, 0)
        pltpu.sync_copy(out_v, out_hbm.at[pl.ds(wid * bpw, bpw)])

    return k(text2, table2)


def _mlp(x, W1, b1, W2, b2):
    B, D = x.shape
    C = W2.shape[0]
    BM = 512
    assert B % BM == 0

    def body(x_ref, w1_ref, b1_ref, w2_ref, b2_ref, o_ref):
        h = lax.dot_general(
            x_ref[...], w1_ref[...], (((1,), (1,)), ((), ())),
            preferred_element_type=jnp.float32,
        )
        h = jnp.maximum(h + b1_ref[...], 0.0)
        o_ref[...] = lax.dot_general(
            h, w2_ref[...], (((1,), (1,)), ((), ())),
            preferred_element_type=jnp.float32,
        ) + b2_ref[...]

    return pl.pallas_call(
        body,
        grid=(B // BM,),
        in_specs=[
            pl.BlockSpec((BM, D), lambda i: (i, 0)),
            pl.BlockSpec((D, D), lambda i: (0, 0)),
            pl.BlockSpec((1, D), lambda i: (0, 0)),
            pl.BlockSpec((C, D), lambda i: (0, 0)),
            pl.BlockSpec((1, C), lambda i: (0, 0)),
        ],
        out_specs=pl.BlockSpec((BM, C), lambda i: (i, 0)),
        out_shape=jax.ShapeDtypeStruct((B, C), jnp.float32),
    )(x, W1, b1.reshape(1, D), W2, b2.reshape(1, C))


def kernel(text, offsets, table, W1, b1, W2, b2):
    del offsets
    B, BAG = text.shape
    D = table.shape[1]
    table2 = _detile(table.T)
    text2 = text.reshape(B // 2, 2 * BAG)
    pooled = _embed_bag_mean(text2, table2, B, BAG, D)
    return _mlp(pooled, W1, b1, W2, b2)


# speedup vs baseline: 1.5029x; 1.0567x over previous
import functools

import jax
import jax.numpy as jnp
from jax import lax
from jax.experimental import pallas as pl
from jax.experimental.pallas import tpu as pltpu
from jax.experimental.pallas import tpu_sc as plsc

_NUM_CORES = 2
_NUM_SUBCORES = 16
_NUM_WORKERS = _NUM_CORES * _NUM_SUBCORES
_LANES = 16


def _detile(tableT):
    D, V = tableT.shape
    BM = 16384
    grid = (V + BM - 1) // BM

    def body(t_ref, o_ref):
        o_ref[:, 0:D] = lax.transpose(t_ref[...], (1, 0))

    return pl.pallas_call(
        body,
        grid=(grid,),
        in_specs=[pl.BlockSpec((D, BM), lambda i: (0, i))],
        out_specs=pl.BlockSpec((BM, 2 * D), lambda i: (i, 0)),
        out_shape=jax.ShapeDtypeStruct((V, 2 * D), jnp.float32),
    )(tableT)


def _embed_bag_mean(text2, table2, B, BAG, D):
    npair = B // 2
    assert npair % _NUM_WORKERS == 0
    ppw = npair // _NUM_WORKERS
    bpw = 2 * ppw
    nch = D // _LANES
    PB = 2 * BAG

    mesh = plsc.VectorSubcoreMesh(core_axis_name="c", subcore_axis_name="s")

    @functools.partial(
        pl.kernel,
        out_type=jax.ShapeDtypeStruct((B, D), jnp.float32),
        mesh=mesh,
        scratch_types=[
            pltpu.VMEM((ppw, PB), jnp.int32),
            pltpu.VMEM((PB, 2 * D), jnp.float32),
            pltpu.VMEM((PB, 2 * D), jnp.float32),
            pltpu.VMEM((bpw, D), jnp.float32),
            pltpu.SemaphoreType.DMA,
            pltpu.SemaphoreType.DMA,
        ],
    )
    def k(text_hbm, table_hbm, out_hbm, idx_v, rows0, rows1, out_v, sem0, sem1):
        wid = lax.axis_index("s") * _NUM_CORES + lax.axis_index("c")
        base = wid * ppw
        pltpu.sync_copy(text_hbm.at[pl.ds(base, ppw)], idx_v)

        scale = jnp.float32(1.0 / BAG)

        def accum(p, rows_v):
            def j_body(t, accs):
                new = list(accs)
                for u in range(2):
                    j = 2 * t + u
                    for h in range(2):
                        for c in range(nch):
                            new[h * nch + c] = (
                                new[h * nch + c]
                                + rows_v[h * BAG + j, pl.ds(c * _LANES, _LANES)]
                            )
                return tuple(new)

            accs = lax.fori_loop(
                0, BAG // 2, j_body,
                tuple(jnp.zeros((_LANES,), jnp.float32) for _ in range(2 * nch)),
            )
            for h in range(2):
                for c in range(nch):
                    out_v[2 * p + h, pl.ds(c * _LANES, _LANES)] = (
                        accs[h * nch + c] * scale
                    )

        pltpu.async_copy(table_hbm.at[idx_v.at[0]], rows0, sem0)
        pltpu.async_copy(table_hbm.at[idx_v.at[1]], rows1, sem1)

        def pair_body(i, carry):
            p0 = 2 * i
            pltpu.make_async_copy(table_hbm.at[idx_v.at[p0]], rows0, sem0).wait()
            accum(p0, rows0)

            @pl.when(i < ppw // 2 - 1)
            def _():
                pltpu.async_copy(table_hbm.at[idx_v.at[p0 + 2]], rows0, sem0)

            p1 = p0 + 1
            pltpu.make_async_copy(table_hbm.at[idx_v.at[p1]], rows1, sem1).wait()
            accum(p1, rows1)

            @pl.when(i < ppw // 2 - 1)
            def _():
                pltpu.async_copy(table_hbm.at[idx_v.at[p1 + 2]], rows1, sem1)

            return carry

        lax.fori_loop(0, ppw // 2, pair_body, 0)
        pltpu.sync_copy(out_v, out_hbm.at[pl.ds(wid * bpw, bpw)])

    return k(text2, table2)


def _mlp(x, W1, b1, W2, b2):
    B, D = x.shape
    C = W2.shape[0]
    BM = 512
    assert B % BM == 0

    def body(x_ref, w1_ref, b1_ref, w2_ref, b2_ref, o_ref):
        h = lax.dot_general(
            x_ref[...], w1_ref[...], (((1,), (1,)), ((), ())),
            preferred_element_type=jnp.float32,
        )
        h = jnp.maximum(h + b1_ref[...], 0.0)
        o_ref[...] = lax.dot_general(
            h, w2_ref[...], (((1,), (1,)), ((), ())),
            preferred_element_type=jnp.float32,
        ) + b2_ref[...]

    return pl.pallas_call(
        body,
        grid=(B // BM,),
        in_specs=[
            pl.BlockSpec((BM, D), lambda i: (i, 0)),
            pl.BlockSpec((D, D), lambda i: (0, 0)),
            pl.BlockSpec((1, D), lambda i: (0, 0)),
            pl.BlockSpec((C, D), lambda i: (0, 0)),
            pl.BlockSpec((1, C), lambda i: (0, 0)),
        ],
        out_specs=pl.BlockSpec((BM, C), lambda i: (i, 0)),
        out_shape=jax.ShapeDtypeStruct((B, C), jnp.float32),
    )(x, W1, b1.reshape(1, D), W2, b2.reshape(1, C))


def kernel(text, offsets, table, W1, b1, W2, b2):
    del offsets
    B, BAG = text.shape
    D = table.shape[1]
    table2 = _detile(table.T)
    text2 = text.reshape(B // 2, 2 * BAG)
    pooled = _embed_bag_mean(text2, table2, B, BAG, D)
    return _mlp(pooled, W1, b1, W2, b2)


# speedup vs baseline: 1.5358x; 1.0219x over previous
import functools

import jax
import jax.numpy as jnp
from jax import lax
from jax.experimental import pallas as pl
from jax.experimental.pallas import tpu as pltpu
from jax.experimental.pallas import tpu_sc as plsc

_NUM_CORES = 2
_NUM_SUBCORES = 16
_NUM_WORKERS = _NUM_CORES * _NUM_SUBCORES
_LANES = 16


def _detile(tableT):
    D, V = tableT.shape
    BM = 32768
    grid = (V + BM - 1) // BM

    def body(t_ref, o_ref):
        o_ref[:, 0:D] = lax.transpose(t_ref[...], (1, 0))

    return pl.pallas_call(
        body,
        grid=(grid,),
        in_specs=[pl.BlockSpec((D, BM), lambda i: (0, i))],
        out_specs=pl.BlockSpec((BM, 2 * D), lambda i: (i, 0)),
        out_shape=jax.ShapeDtypeStruct((V, 2 * D), jnp.float32),
    )(tableT)


def _embed_bag_mean(text2, table2, B, BAG, D):
    npair = B // 2
    assert npair % _NUM_WORKERS == 0
    ppw = npair // _NUM_WORKERS
    bpw = 2 * ppw
    nch = D // _LANES
    PB = 2 * BAG

    mesh = plsc.VectorSubcoreMesh(core_axis_name="c", subcore_axis_name="s")

    @functools.partial(
        pl.kernel,
        out_type=jax.ShapeDtypeStruct((B, D), jnp.float32),
        mesh=mesh,
        scratch_types=[
            pltpu.VMEM((ppw, PB), jnp.int32),
            pltpu.VMEM((PB, 2 * D), jnp.float32),
            pltpu.VMEM((PB, 2 * D), jnp.float32),
            pltpu.VMEM((bpw, D), jnp.float32),
            pltpu.SemaphoreType.DMA,
            pltpu.SemaphoreType.DMA,
        ],
    )
    def k(text_hbm, table_hbm, out_hbm, idx_v, rows0, rows1, out_v, sem0, sem1):
        wid = lax.axis_index("s") * _NUM_CORES + lax.axis_index("c")
        base = wid * ppw
        pltpu.sync_copy(text_hbm.at[pl.ds(base, ppw)], idx_v)

        scale = jnp.float32(1.0 / BAG)

        def accum(p, rows_v):
            def j_body(t, accs):
                new = list(accs)
                for u in range(2):
                    j = 2 * t + u
                    for h in range(2):
                        for c in range(nch):
                            new[h * nch + c] = (
                                new[h * nch + c]
                                + rows_v[h * BAG + j, pl.ds(c * _LANES, _LANES)]
                            )
                return tuple(new)

            accs = lax.fori_loop(
                0, BAG // 2, j_body,
                tuple(jnp.zeros((_LANES,), jnp.float32) for _ in range(2 * nch)),
            )
            for h in range(2):
                for c in range(nch):
                    out_v[2 * p + h, pl.ds(c * _LANES, _LANES)] = (
                        accs[h * nch + c] * scale
                    )

        pltpu.async_copy(table_hbm.at[idx_v.at[0]], rows0, sem0)
        pltpu.async_copy(table_hbm.at[idx_v.at[1]], rows1, sem1)

        def pair_body(i, carry):
            p0 = 2 * i
            pltpu.make_async_copy(table_hbm.at[idx_v.at[p0]], rows0, sem0).wait()
            accum(p0, rows0)

            @pl.when(i < ppw // 2 - 1)
            def _():
                pltpu.async_copy(table_hbm.at[idx_v.at[p0 + 2]], rows0, sem0)

            p1 = p0 + 1
            pltpu.make_async_copy(table_hbm.at[idx_v.at[p1]], rows1, sem1).wait()
            accum(p1, rows1)

            @pl.when(i < ppw // 2 - 1)
            def _():
                pltpu.async_copy(table_hbm.at[idx_v.at[p1 + 2]], rows1, sem1)

            return carry

        lax.fori_loop(0, ppw // 2, pair_body, 0)
        pltpu.sync_copy(out_v, out_hbm.at[pl.ds(wid * bpw, bpw)])

    return k(text2, table2)


def _mlp(x, W1, b1, W2, b2):
    B, D = x.shape
    C = W2.shape[0]
    BM = 512
    assert B % BM == 0

    def body(x_ref, w1_ref, b1_ref, w2_ref, b2_ref, o_ref):
        h = lax.dot_general(
            x_ref[...], w1_ref[...], (((1,), (1,)), ((), ())),
            preferred_element_type=jnp.float32,
        )
        h = jnp.maximum(h + b1_ref[...], 0.0)
        o_ref[...] = lax.dot_general(
            h, w2_ref[...], (((1,), (1,)), ((), ())),
            preferred_element_type=jnp.float32,
        ) + b2_ref[...]

    return pl.pallas_call(
        body,
        grid=(B // BM,),
        in_specs=[
            pl.BlockSpec((BM, D), lambda i: (i, 0)),
            pl.BlockSpec((D, D), lambda i: (0, 0)),
            pl.BlockSpec((1, D), lambda i: (0, 0)),
            pl.BlockSpec((C, D), lambda i: (0, 0)),
            pl.BlockSpec((1, C), lambda i: (0, 0)),
        ],
        out_specs=pl.BlockSpec((BM, C), lambda i: (i, 0)),
        out_shape=jax.ShapeDtypeStruct((B, C), jnp.float32),
    )(x, W1, b1.reshape(1, D), W2, b2.reshape(1, C))


def kernel(text, offsets, table, W1, b1, W2, b2):
    del offsets
    B, BAG = text.shape
    D = table.shape[1]
    table2 = _detile(table.T)
    text2 = text.reshape(B // 2, 2 * BAG)
    pooled = _embed_bag_mean(text2, table2, B, BAG, D)
    return _mlp(pooled, W1, b1, W2, b2)


# speedup vs baseline: 1.5365x; 1.0005x over previous
import functools

import jax
import jax.numpy as jnp
from jax import lax
from jax.experimental import pallas as pl
from jax.experimental.pallas import tpu as pltpu
from jax.experimental.pallas import tpu_sc as plsc

_NUM_CORES = 2
_NUM_SUBCORES = 16
_NUM_WORKERS = _NUM_CORES * _NUM_SUBCORES
_LANES = 16


def _detile(tableT):
    D, V = tableT.shape
    BM = 32768
    grid = (V + BM - 1) // BM

    def body(t_ref, o_ref):
        o_ref[:, 0:D] = lax.transpose(t_ref[...], (1, 0))

    return pl.pallas_call(
        body,
        grid=(grid,),
        in_specs=[pl.BlockSpec((D, BM), lambda i: (0, i))],
        out_specs=pl.BlockSpec((BM, 2 * D), lambda i: (i, 0)),
        out_shape=jax.ShapeDtypeStruct((V, 2 * D), jnp.float32),
    )(tableT)


def _embed_bag_mean(text2, table2, B, BAG, D):
    npair = B // 2
    assert npair % _NUM_WORKERS == 0
    ppw = npair // _NUM_WORKERS
    bpw = 2 * ppw
    nch = D // _LANES
    PB = 2 * BAG

    mesh = plsc.VectorSubcoreMesh(core_axis_name="c", subcore_axis_name="s")

    @functools.partial(
        pl.kernel,
        out_type=jax.ShapeDtypeStruct((B, D), jnp.float32),
        mesh=mesh,
        scratch_types=[
            pltpu.VMEM((ppw, PB), jnp.int32),
            pltpu.VMEM((PB, 2 * D), jnp.float32),
            pltpu.VMEM((PB, 2 * D), jnp.float32),
            pltpu.VMEM((bpw, D), jnp.float32),
            pltpu.SemaphoreType.DMA,
            pltpu.SemaphoreType.DMA,
        ],
    )
    def k(text_hbm, table_hbm, out_hbm, idx_v, rows0, rows1, out_v, sem0, sem1):
        wid = lax.axis_index("s") * _NUM_CORES + lax.axis_index("c")
        base = wid * ppw
        pltpu.sync_copy(text_hbm.at[pl.ds(base, ppw)], idx_v)

        scale = jnp.float32(1.0 / BAG)

        def accum(p, rows_v):
            UNROLL = 5

            def j_body(t, accs):
                new = list(accs)
                for u in range(UNROLL):
                    j = UNROLL * t + u
                    for h in range(2):
                        for c in range(nch):
                            new[h * nch + c] = (
                                new[h * nch + c]
                                + rows_v[h * BAG + j, pl.ds(c * _LANES, _LANES)]
                            )
                return tuple(new)

            accs = lax.fori_loop(
                0, BAG // UNROLL, j_body,
                tuple(jnp.zeros((_LANES,), jnp.float32) for _ in range(2 * nch)),
            )
            for h in range(2):
                for c in range(nch):
                    out_v[2 * p + h, pl.ds(c * _LANES, _LANES)] = (
                        accs[h * nch + c] * scale
                    )

        pltpu.async_copy(table_hbm.at[idx_v.at[0]], rows0, sem0)
        pltpu.async_copy(table_hbm.at[idx_v.at[1]], rows1, sem1)

        def pair_body(i, carry):
            p0 = 2 * i
            pltpu.make_async_copy(table_hbm.at[idx_v.at[p0]], rows0, sem0).wait()
            accum(p0, rows0)

            @pl.when(i < ppw // 2 - 1)
            def _():
                pltpu.async_copy(table_hbm.at[idx_v.at[p0 + 2]], rows0, sem0)

            p1 = p0 + 1
            pltpu.make_async_copy(table_hbm.at[idx_v.at[p1]], rows1, sem1).wait()
            accum(p1, rows1)

            @pl.when(i < ppw // 2 - 1)
            def _():
                pltpu.async_copy(table_hbm.at[idx_v.at[p1 + 2]], rows1, sem1)

            return carry

        lax.fori_loop(0, ppw // 2, pair_body, 0)
        pltpu.sync_copy(out_v, out_hbm.at[pl.ds(wid * bpw, bpw)])

    return k(text2, table2)


def _mlp(x, W1, b1, W2, b2):
    B, D = x.shape
    C = W2.shape[0]
    BM = 512
    assert B % BM == 0

    def body(x_ref, w1_ref, b1_ref, w2_ref, b2_ref, o_ref):
        h = lax.dot_general(
            x_ref[...], w1_ref[...], (((1,), (1,)), ((), ())),
            preferred_element_type=jnp.float32,
        )
        h = jnp.maximum(h + b1_ref[...], 0.0)
        o_ref[...] = lax.dot_general(
            h, w2_ref[...], (((1,), (1,)), ((), ())),
            preferred_element_type=jnp.float32,
        ) + b2_ref[...]

    return pl.pallas_call(
        body,
        grid=(B // BM,),
        in_specs=[
            pl.BlockSpec((BM, D), lambda i: (i, 0)),
            pl.BlockSpec((D, D), lambda i: (0, 0)),
            pl.BlockSpec((1, D), lambda i: (0, 0)),
            pl.BlockSpec((C, D), lambda i: (0, 0)),
            pl.BlockSpec((1, C), lambda i: (0, 0)),
        ],
        out_specs=pl.BlockSpec((BM, C), lambda i: (i, 0)),
        out_shape=jax.ShapeDtypeStruct((B, C), jnp.float32),
    )(x, W1, b1.reshape(1, D), W2, b2.reshape(1, C))


def kernel(text, offsets, table, W1, b1, W2, b2):
    del offsets
    B, BAG = text.shape
    D = table.shape[1]
    table2 = _detile(table.T)
    text2 = text.reshape(B // 2, 2 * BAG)
    pooled = _embed_bag_mean(text2, table2, B, BAG, D)
    return _mlp(pooled, W1, b1, W2, b2)
